# Initial kernel scaffold; baseline (speedup 1.0000x reference)
import jax
import jax.numpy as jnp
from jax import lax
from jax.experimental import pallas as pl
from jax.experimental.pallas import tpu as pltpu
from jax.experimental.pallas import tpu_sc as plsc

N = 10000
F = 256
FH = 128
E = 160000
NSUB = 16
NCORE = 2
EPT = E // NSUB
K = 80
NCH = EPT // K
RPT = N // NSUB
DR = 25
NP = 10240
SPT = NP // NSUB


def _mesh():
  return plsc.VectorSubcoreMesh(core_axis_name="c", subcore_axis_name="s")


def _fill_vmem(ref, nwords, value):
  z = jnp.full((16,), value, jnp.float32)

  @pl.loop(0, nwords // 16)
  def _(i):
    ref[pl.ds(i * 16, 16)] = z


def _rsqrt_sc(x):
  bits = plsc.bitcast(x, jnp.int32)
  y = plsc.bitcast(jnp.int32(0x5F3759DF) - (bits >> 1), jnp.float32)
  for _ in range(3):
    y = y * (1.5 - 0.5 * x * y * y)
  return y


def _recip_or_zero(c):
  safe = jnp.where(c > 0.0, c, 1.0)
  return jnp.where(c > 0.0, 1.0 / safe, 0.0)


def _counts_body(dst3, node3, he3, dinv_out, Dinv_out, Binv_out,
                 idxbuf, ones, cbuf, obuf, obuf2, accA, accB, sem):
  del sem
  cid = lax.axis_index("c")
  sid = lax.axis_index("s")

  _fill_vmem(ones, K * 16, 1.0)
  _fill_vmem(cbuf, SPT * 16, 0.0)
  c2 = cbuf.reshape(SPT, 16)
  o2 = ones.reshape(K, 16)

  pltpu.sync_copy(c2, accA.at[pl.ds(sid * SPT, SPT)])
  pltpu.sync_copy(c2, accB.at[pl.ds(sid * SPT, SPT)])
  plsc.subcore_barrier()

  def hist(idx3_hbm, acc):
    pltpu.sync_copy(idx3_hbm.at[sid], idxbuf)

    @pl.loop(0, NCH)
    def _(j):
      pltpu.sync_copy(o2, acc.at[idxbuf.at[j]], add=True)

  @pl.when(cid == 0)
  def _():
    hist(dst3, accA)

  @pl.when(cid == 1)
  def _():
    hist(node3, accA)
    hist(he3, accB)

  plsc.subcore_barrier()

  base = sid * SPT
  lane = lax.iota(jnp.int32, 16)
  zero16 = jnp.zeros((16,), jnp.int32)

  def drain(acc, out_hbm, transform, outbuf):
    pltpu.sync_copy(acc.at[pl.ds(base, SPT)], c2)

    @pl.loop(0, SPT // 16)
    def _(g):
      cnt = plsc.load_gather(c2, [g * 16 + lane, zero16])
      outbuf[pl.ds(g * 16, 16)] = transform(cnt)

    pltpu.sync_copy(outbuf, out_hbm.at[pl.ds(base, SPT)])

  @pl.when(cid == 0)
  def _():
    drain(accA, dinv_out, lambda c: _rsqrt_sc(c + 1.0), obuf)

  @pl.when(cid == 1)
  def _():
    drain(accA, Dinv_out, _recip_or_zero, obuf)
    drain(accB, Binv_out, _recip_or_zero, obuf2)


def _make_counts():
  out = (jax.ShapeDtypeStruct((NP,), jnp.float32),) * 3
  return pl.kernel(
      _counts_body,
      out_type=out,
      mesh=_mesh(),
      scratch_types=[
          pltpu.VMEM((NCH, K), jnp.int32),
          pltpu.VMEM((K * 16,), jnp.float32),
          pltpu.VMEM((SPT * 16,), jnp.float32),
          pltpu.VMEM((SPT,), jnp.float32),
          pltpu.VMEM((SPT,), jnp.float32),
          pltpu.VMEM_SHARED((NP, 16), jnp.float32),
          pltpu.VMEM_SHARED((NP, 16), jnp.float32),
          pltpu.SemaphoreType.DMA,
      ],
  )


def _make_agg(scale: bool):
  def body(*refs):
    if scale:
      (x2, g3, s3, svec, out2, gbuf, sbuf, rows, zbuf, dbuf, scbuf, acc,
       sem) = refs
    else:
      (x2, g3, s3, out2, gbuf, sbuf, rows, zbuf, dbuf, acc, sem) = refs
      svec = scbuf = None
    cid = lax.axis_index("c")
    sid = lax.axis_index("s")

    _fill_vmem(zbuf, DR * FH, 0.0)
    z2 = zbuf.reshape(DR, FH)
    rows_base = sid * RPT

    @pl.loop(0, RPT // DR)
    def _(j):
      pltpu.sync_copy(z2, acc.at[pl.ds(rows_base + j * DR, DR)])

    pltpu.sync_copy(g3.at[sid], gbuf)
    pltpu.sync_copy(s3.at[sid], sbuf)
    if scale:
      pltpu.sync_copy(svec, scbuf)
    plsc.subcore_barrier()

    def edge_loop(xv):
      @pl.loop(0, NCH)
      def _(j):
        pltpu.async_copy(xv.at[gbuf.at[j]], rows, sem).wait()
        pltpu.sync_copy(rows, acc.at[sbuf.at[j]], add=True)

    @pl.when(cid == 0)
    def _():
      edge_loop(x2.at[0])

    @pl.when(cid == 1)
    def _():
      edge_loop(x2.at[1])

    plsc.subcore_barrier()

    def drain(outv):
      @pl.loop(0, RPT // DR)
      def _(k):
        r0 = rows_base + k * DR
        if not scale:
          pltpu.sync_copy(acc.at[pl.ds(r0, DR)], outv.at[pl.ds(r0, DR)])
        else:
          pltpu.sync_copy(acc.at[pl.ds(r0, DR)], dbuf)

          @pl.loop(0, DR)
          def _(r):
            v = scbuf[r0 + r]
            for q in range(FH // 16):
              dbuf[r, pl.ds(q * 16, 16)] = dbuf[r, pl.ds(q * 16, 16)] * v

          pltpu.sync_copy(dbuf, outv.at[pl.ds(r0, DR)])

    @pl.when(cid == 0)
    def _():
      drain(out2.at[0])

    @pl.when(cid == 1)
    def _():
      drain(out2.at[1])

  scratch = [
      pltpu.VMEM((NCH, K), jnp.int32),
      pltpu.VMEM((NCH, K), jnp.int32),
      pltpu.VMEM((K, FH), jnp.float32),
      pltpu.VMEM((DR * FH,), jnp.float32),
      pltpu.VMEM((DR, FH), jnp.float32),
  ]
  if scale:
    scratch.append(pltpu.VMEM((N,), jnp.float32))
  scratch += [
      pltpu.VMEM_SHARED((N, FH), jnp.float32),
      pltpu.SemaphoreType.DMA,
  ]
  return pl.kernel(
      body,
      out_type=jax.ShapeDtypeStruct((NCORE, N, FH), jnp.float32),
      mesh=_mesh(),
      scratch_types=scratch,
  )


BR = 2000
_NBLK = N // BR


def _tc_call(body, in_specs, out_specs, out_shape):
  return pl.pallas_call(
      body,
      grid=(_NBLK,),
      in_specs=in_specs,
      out_specs=out_specs,
      out_shape=out_shape,
  )


def _row_spec(w):
  return pl.BlockSpec((BR, w), lambda i: (i, 0))


def _split_spec():
  return pl.BlockSpec((NCORE, BR, FH), lambda i: (0, i, 0))


def _full_spec(shape):
  return pl.BlockSpec(shape, lambda i: tuple(0 for _ in shape))


def _cat(ref):
  return jnp.concatenate([ref[0], ref[1]], axis=-1)


def _prep_body(emb_ref, dinv_ref, xs_ref):
  x = emb_ref[...] * dinv_ref[...]
  xs_ref[0] = x[:, :FH]
  xs_ref[1] = x[:, FH:]


def _prep(emb, dinv_c):
  return _tc_call(
      _prep_body,
      [_row_spec(F), _row_spec(1)],
      _split_spec(),
      jax.ShapeDtypeStruct((NCORE, N, FH), jnp.float32),
  )(emb, dinv_c)


def _mid_body(agg_ref, xs_ref, dinv_ref, W1_ref, b1_ref, W2_ref, ts_ref):
  dinv = dinv_ref[...]
  u = dinv * (_cat(agg_ref) + _cat(xs_ref))
  h1 = jnp.dot(u, W1_ref[...], preferred_element_type=jnp.float32) + b1_ref[...]
  t = jnp.dot(h1, W2_ref[...], preferred_element_type=jnp.float32)
  ts = t * dinv
  ts_ref[0] = ts[:, :FH]
  ts_ref[1] = ts[:, FH:]


def _mid(agg1, xs2, dinv_c, W1, b1r, W2):
  return _tc_call(
      _mid_body,
      [_split_spec(), _split_spec(), _row_spec(1),
       _full_spec((F, 2 * F)), _full_spec((1, 2 * F)), _full_spec((2 * F, F))],
      _split_spec(),
      jax.ShapeDtypeStruct((NCORE, N, FH), jnp.float32),
  )(agg1, xs2, dinv_c, W1, b1r, W2)


def _heter_body(agg_ref, ts_ref, dinv_ref, b2_ref, ho_ref, xh_ref):
  ho = dinv_ref[...] * (_cat(agg_ref) + _cat(ts_ref)) + b2_ref[...]
  ho_ref[...] = ho
  xh_ref[0] = ho[:, :FH]
  xh_ref[1] = ho[:, FH:]


def _heter(agg2, ts2, dinv_c, b2r):
  return _tc_call(
      _heter_body,
      [_split_spec(), _split_spec(), _row_spec(1), _full_spec((1, F))],
      [_row_spec(F), _split_spec()],
      (jax.ShapeDtypeStruct((N, F), jnp.float32),
       jax.ShapeDtypeStruct((NCORE, N, FH), jnp.float32)),
  )(agg2, ts2, dinv_c, b2r)


def _hyper_body(hp_ref, Dinv_ref, W4_ref, b4_ref, out_ref):
  dinv = Dinv_ref[...]
  xa = hp_ref[0] * dinv
  xb = hp_ref[1] * dinv
  W4 = W4_ref[...]
  out_ref[...] = (
      jnp.dot(xa, W4[:FH, :], preferred_element_type=jnp.float32)
      + jnp.dot(xb, W4[FH:, :], preferred_element_type=jnp.float32)
      + b4_ref[...])


def _hyper_tc(hp2, Dinv_c, W4, b4r):
  return _tc_call(
      _hyper_body,
      [_split_spec(), _row_spec(1), _full_spec((F, F)), _full_spec((1, F))],
      _row_spec(F),
      jax.ShapeDtypeStruct((N, F), jnp.float32),
  )(hp2, Dinv_c, W4, b4r)


_counts = _make_counts()
_agg_plain = _make_agg(scale=False)
_agg_scaled = _make_agg(scale=True)


def kernel(heter_edge_index, heter_edge_type, hyper_edge_index, emb,
           W1, b1, W2, b2, W4, b4):
  del heter_edge_type
  src3 = heter_edge_index[0].reshape(NSUB, NCH, K)
  dst3 = heter_edge_index[1].reshape(NSUB, NCH, K)
  node3 = hyper_edge_index[0].reshape(NSUB, NCH, K)
  he3 = hyper_edge_index[1].reshape(NSUB, NCH, K)

  dinv_p, Dinv_p, Binv_p = _counts(dst3, node3, he3)
  dinv_c = dinv_p[:N].reshape(N, 1)
  Dinv_c = Dinv_p[:N].reshape(N, 1)
  Binv = Binv_p[:N]

  xs2 = _prep(emb, dinv_c)
  agg1 = _agg_plain(xs2, src3, dst3)
  ts2 = _mid(agg1, xs2, dinv_c, W1, b1.reshape(1, -1), W2)
  agg2 = _agg_plain(ts2, src3, dst3)
  heter_out, xh2 = _heter(agg2, ts2, dinv_c, b2.reshape(1, -1))

  ef2 = _agg_scaled(xh2, node3, he3, Binv)
  hp2 = _agg_plain(ef2, he3, node3)
  hyper_out = _hyper_tc(hp2, Dinv_c, W4, b4.reshape(1, -1))
  return heter_out, hyper_out


# speedup vs baseline: 8.1545x; 8.1545x over previous
import jax
import jax.numpy as jnp
from jax import lax
from jax.experimental import pallas as pl
from jax.experimental.pallas import tpu as pltpu
from jax.experimental.pallas import tpu_sc as plsc

N = 10000
F = 256
FH = 128
E = 160000
NSUB = 16
NCORE = 2
EPT = E // NSUB
K = 80
NCH = EPT // K
DR = 16
NP = 10240
SPT = NP // NSUB


def _mesh():
  return plsc.VectorSubcoreMesh(core_axis_name="c", subcore_axis_name="s")


def _fill_vmem2(ref, nrows, ncols, value):
  z = jnp.full((16,), value, jnp.float32)

  @pl.loop(0, nrows)
  def _(i):
    for q in range(ncols // 16):
      ref[i, pl.ds(q * 16, 16)] = z


def _counts_body(dst3, node3, he3, deg_out, D_out, B_out,
                 idxbuf, ones, zbuf, acc, sem):
  del sem
  cid = lax.axis_index("c")
  sid = lax.axis_index("s")

  _fill_vmem2(ones, K, FH, 1.0)
  _fill_vmem2(zbuf, DR, FH, 0.0)
  base = sid * SPT

  def zero_acc():
    @pl.loop(0, SPT // DR)
    def _(j):
      pltpu.sync_copy(zbuf, acc.at[pl.ds(base + j * DR, DR)])

  def hist(idx3_hbm):
    pltpu.sync_copy(idx3_hbm.at[sid], idxbuf)

    @pl.loop(0, NCH)
    def _(j):
      pltpu.sync_copy(ones, acc.at[idxbuf.at[j]], add=True)

  def drain(out_hbm):
    pltpu.sync_copy(acc.at[pl.ds(base, SPT)], out_hbm.at[pl.ds(base, SPT)])

  zero_acc()
  plsc.subcore_barrier()

  @pl.when(cid == 0)
  def _():
    hist(dst3)

  @pl.when(cid == 1)
  def _():
    hist(node3)

  plsc.subcore_barrier()

  @pl.when(cid == 0)
  def _():
    drain(deg_out)

  @pl.when(cid == 1)
  def _():
    drain(D_out)

  zero_acc()
  plsc.subcore_barrier()

  @pl.when(cid == 1)
  def _():
    hist(he3)

  plsc.subcore_barrier()

  @pl.when(cid == 1)
  def _():
    drain(B_out)


def _make_counts():
  out = (jax.ShapeDtypeStruct((NP, FH), jnp.float32),) * 3
  return pl.kernel(
      _counts_body,
      out_type=out,
      mesh=_mesh(),
      scratch_types=[
          pltpu.VMEM((NCH, K), jnp.int32),
          pltpu.VMEM((K, FH), jnp.float32),
          pltpu.VMEM((DR, FH), jnp.float32),
          pltpu.VMEM_SHARED((NP, FH), jnp.float32),
          pltpu.SemaphoreType.DMA,
      ],
  )


def _make_agg():
  def body(x2, g3, s3, out2, gbuf, sbuf, rows, zbuf, acc, sem):
    cid = lax.axis_index("c")
    sid = lax.axis_index("s")

    _fill_vmem2(zbuf, DR, FH, 0.0)
    rows_base = sid * SPT

    @pl.loop(0, SPT // DR)
    def _(j):
      pltpu.sync_copy(zbuf, acc.at[pl.ds(rows_base + j * DR, DR)])

    pltpu.sync_copy(g3.at[sid], gbuf)
    pltpu.sync_copy(s3.at[sid], sbuf)
    plsc.subcore_barrier()

    def edge_loop(xv):
      @pl.loop(0, NCH)
      def _(j):
        pltpu.async_copy(xv.at[gbuf.at[j]], rows, sem).wait()
        pltpu.sync_copy(rows, acc.at[sbuf.at[j]], add=True)

    @pl.when(cid == 0)
    def _():
      edge_loop(x2.at[0])

    @pl.when(cid == 1)
    def _():
      edge_loop(x2.at[1])

    plsc.subcore_barrier()

    ntrips = jnp.where(sid == NSUB - 1, (N - (NSUB - 1) * SPT) // DR,
                       SPT // DR)

    def drain(outv):
      @pl.loop(0, ntrips)
      def _(k):
        r0 = rows_base + k * DR
        pltpu.sync_copy(acc.at[pl.ds(r0, DR)], outv.at[pl.ds(r0, DR)])

    @pl.when(cid == 0)
    def _():
      drain(out2.at[0])

    @pl.when(cid == 1)
    def _():
      drain(out2.at[1])

  scratch = [
      pltpu.VMEM((NCH, K), jnp.int32),
      pltpu.VMEM((NCH, K), jnp.int32),
      pltpu.VMEM((K, FH), jnp.float32),
      pltpu.VMEM((DR, FH), jnp.float32),
      pltpu.VMEM_SHARED((NP, FH), jnp.float32),
      pltpu.SemaphoreType.DMA,
  ]
  return pl.kernel(
      body,
      out_type=jax.ShapeDtypeStruct((NCORE, N, FH), jnp.float32),
      mesh=_mesh(),
      scratch_types=scratch,
  )


BR = 2000
_NBLK = N // BR


def _tc_call(body, in_specs, out_specs, out_shape):
  return pl.pallas_call(
      body,
      grid=(_NBLK,),
      in_specs=in_specs,
      out_specs=out_specs,
      out_shape=out_shape,
  )


def _row_spec(w):
  return pl.BlockSpec((BR, w), lambda i: (i, 0))


def _split_spec():
  return pl.BlockSpec((NCORE, BR, FH), lambda i: (0, i, 0))


def _full_spec(shape):
  return pl.BlockSpec(shape, lambda i: tuple(0 for _ in shape))


def _cat(ref):
  return jnp.concatenate([ref[0], ref[1]], axis=-1)


def _prep_body(emb_ref, deg_ref, xs_ref, dinv_ref):
  dinv = lax.rsqrt(deg_ref[...] + 1.0)
  x = emb_ref[...] * dinv
  xs_ref[0] = x[:, :FH]
  xs_ref[1] = x[:, FH:]
  dinv_ref[...] = dinv


def _prep(emb, deg_c):
  return _tc_call(
      _prep_body,
      [_row_spec(F), _row_spec(1)],
      [_split_spec(), _row_spec(1)],
      (jax.ShapeDtypeStruct((NCORE, N, FH), jnp.float32),
       jax.ShapeDtypeStruct((N, 1), jnp.float32)),
  )(emb, deg_c)


def _mid_body(agg_ref, xs_ref, dinv_ref, W1_ref, b1_ref, W2_ref, ts_ref):
  dinv = dinv_ref[...]
  u = dinv * (_cat(agg_ref) + _cat(xs_ref))
  h1 = jnp.dot(u, W1_ref[...], preferred_element_type=jnp.float32) + b1_ref[...]
  t = jnp.dot(h1, W2_ref[...], preferred_element_type=jnp.float32)
  ts = t * dinv
  ts_ref[0] = ts[:, :FH]
  ts_ref[1] = ts[:, FH:]


def _mid(agg1, xs2, dinv_c, W1, b1r, W2):
  return _tc_call(
      _mid_body,
      [_split_spec(), _split_spec(), _row_spec(1),
       _full_spec((F, 2 * F)), _full_spec((1, 2 * F)), _full_spec((2 * F, F))],
      _split_spec(),
      jax.ShapeDtypeStruct((NCORE, N, FH), jnp.float32),
  )(agg1, xs2, dinv_c, W1, b1r, W2)


def _heter_body(agg_ref, ts_ref, dinv_ref, b2_ref, B_ref, ho_ref, xh_ref,
                binv_ref):
  ho = dinv_ref[...] * (_cat(agg_ref) + _cat(ts_ref)) + b2_ref[...]
  ho_ref[...] = ho
  xh_ref[0] = ho[:, :FH]
  xh_ref[1] = ho[:, FH:]
  b = B_ref[...]
  binv_ref[...] = jnp.where(b > 0.0, 1.0 / jnp.where(b > 0.0, b, 1.0), 0.0)


def _heter(agg2, ts2, dinv_c, b2r, Bc):
  return _tc_call(
      _heter_body,
      [_split_spec(), _split_spec(), _row_spec(1), _full_spec((1, F)),
       _row_spec(1)],
      [_row_spec(F), _split_spec(), _row_spec(1)],
      (jax.ShapeDtypeStruct((N, F), jnp.float32),
       jax.ShapeDtypeStruct((NCORE, N, FH), jnp.float32),
       jax.ShapeDtypeStruct((N, 1), jnp.float32)),
  )(agg2, ts2, dinv_c, b2r, Bc)


def _bscale_body(ef_ref, binv_ref, out_ref):
  binv = binv_ref[...]
  out_ref[0] = ef_ref[0] * binv
  out_ref[1] = ef_ref[1] * binv


def _bscale(ef2raw, binv_c):
  return _tc_call(
      _bscale_body,
      [_split_spec(), _row_spec(1)],
      _split_spec(),
      jax.ShapeDtypeStruct((NCORE, N, FH), jnp.float32),
  )(ef2raw, binv_c)


def _hyper_body(hp_ref, D_ref, W4_ref, b4_ref, out_ref):
  d = D_ref[...]
  dinv = jnp.where(d > 0.0, 1.0 / jnp.where(d > 0.0, d, 1.0), 0.0)
  xa = hp_ref[0] * dinv
  xb = hp_ref[1] * dinv
  W4 = W4_ref[...]
  out_ref[...] = (
      jnp.dot(xa, W4[:FH, :], preferred_element_type=jnp.float32)
      + jnp.dot(xb, W4[FH:, :], preferred_element_type=jnp.float32)
      + b4_ref[...])


def _hyper_tc(hp2, Dc, W4, b4r):
  return _tc_call(
      _hyper_body,
      [_split_spec(), _row_spec(1), _full_spec((F, F)), _full_spec((1, F))],
      _row_spec(F),
      jax.ShapeDtypeStruct((N, F), jnp.float32),
  )(hp2, Dc, W4, b4r)


_counts = _make_counts()
_agg_plain = _make_agg()


def kernel(heter_edge_index, heter_edge_type, hyper_edge_index, emb,
           W1, b1, W2, b2, W4, b4):
  del heter_edge_type
  src3 = heter_edge_index[0].reshape(NSUB, NCH, K)
  dst3 = heter_edge_index[1].reshape(NSUB, NCH, K)
  node3 = hyper_edge_index[0].reshape(NSUB, NCH, K)
  he3 = hyper_edge_index[1].reshape(NSUB, NCH, K)

  deg_p, D_p, B_p = _counts(dst3, node3, he3)
  deg_c = deg_p[:N, :1]
  Dc = D_p[:N, :1]
  Bc = B_p[:N, :1]

  xs2, dinv_c = _prep(emb, deg_c)
  agg1 = _agg_plain(xs2, src3, dst3)
  ts2 = _mid(agg1, xs2, dinv_c, W1, b1.reshape(1, -1), W2)
  agg2 = _agg_plain(ts2, src3, dst3)
  heter_out, xh2, binv_c = _heter(agg2, ts2, dinv_c, b2.reshape(1, -1), Bc)

  ef2 = _bscale(_agg_plain(xh2, node3, he3), binv_c)
  hp2 = _agg_plain(ef2, he3, node3)
  hyper_out = _hyper_tc(hp2, Dc, W4, b4.reshape(1, -1))
  return heter_out, hyper_out


# speedup vs baseline: 9.1818x; 1.1260x over previous
import jax
import jax.numpy as jnp
from jax import lax
from jax.experimental import pallas as pl
from jax.experimental.pallas import tpu as pltpu
from jax.experimental.pallas import tpu_sc as plsc

N = 10000
F = 256
FH = 128
E = 160000
NSUB = 16
NCORE = 2
EPT = E // NSUB
K = 125
NCH = EPT // K
DR = 16
NP = 10240
SPT = NP // NSUB


def _mesh():
  return plsc.VectorSubcoreMesh(core_axis_name="c", subcore_axis_name="s")


def _fill_vmem2(ref, nrows, ncols, value):
  z = jnp.full((16,), value, jnp.float32)

  @pl.loop(0, nrows)
  def _(i):
    for q in range(ncols // 16):
      ref[i, pl.ds(q * 16, 16)] = z


def _counts_body(dst3, node3, he3, deg_out, D_out, B_out,
                 idxbuf, ones, zbuf, acc, sem):
  del sem
  cid = lax.axis_index("c")
  sid = lax.axis_index("s")

  _fill_vmem2(ones, K, FH, 1.0)
  _fill_vmem2(zbuf, DR, FH, 0.0)
  base = sid * SPT

  def zero_acc():
    @pl.loop(0, SPT // DR)
    def _(j):
      pltpu.sync_copy(zbuf, acc.at[pl.ds(base + j * DR, DR)])

  def hist(idx3_hbm):
    pltpu.sync_copy(idx3_hbm.at[sid], idxbuf)

    @pl.loop(0, NCH)
    def _(j):
      pltpu.sync_copy(ones, acc.at[idxbuf.at[j]], add=True)

  def drain(out_hbm):
    pltpu.sync_copy(acc.at[pl.ds(base, SPT)], out_hbm.at[pl.ds(base, SPT)])

  zero_acc()
  plsc.subcore_barrier()

  @pl.when(cid == 0)
  def _():
    hist(dst3)

  @pl.when(cid == 1)
  def _():
    hist(node3)

  plsc.subcore_barrier()

  @pl.when(cid == 0)
  def _():
    drain(deg_out)

  @pl.when(cid == 1)
  def _():
    drain(D_out)

  zero_acc()
  plsc.subcore_barrier()

  @pl.when(cid == 1)
  def _():
    hist(he3)

  plsc.subcore_barrier()

  @pl.when(cid == 1)
  def _():
    drain(B_out)


def _make_counts():
  out = (jax.ShapeDtypeStruct((NP, FH), jnp.float32),) * 3
  return pl.kernel(
      _counts_body,
      out_type=out,
      mesh=_mesh(),
      scratch_types=[
          pltpu.VMEM((NCH, K), jnp.int32),
          pltpu.VMEM((K, FH), jnp.float32),
          pltpu.VMEM((DR, FH), jnp.float32),
          pltpu.VMEM_SHARED((NP, FH), jnp.float32),
          pltpu.SemaphoreType.DMA,
      ],
  )


def _make_agg():
  def body(x2, g3, s3, out2, gbuf, sbuf, rows0, zbuf, acc, sem0):
    cid = lax.axis_index("c")
    sid = lax.axis_index("s")

    _fill_vmem2(zbuf, DR, FH, 0.0)
    zsrc = zbuf
    rows_base = sid * SPT

    @pl.loop(0, SPT // DR)
    def _(j):
      pltpu.sync_copy(zsrc, acc.at[pl.ds(rows_base + j * DR, DR)])

    pltpu.sync_copy(g3.at[sid], gbuf)
    pltpu.sync_copy(s3.at[sid], sbuf)
    plsc.subcore_barrier()

    def edge_loop(xv):
      @pl.loop(0, NCH)
      def _(j):
        pltpu.async_copy(xv.at[gbuf.at[j]], rows0, sem0).wait()
        pltpu.sync_copy(rows0, acc.at[sbuf.at[j]], add=True)

    @pl.when(cid == 0)
    def _():
      edge_loop(x2.at[0])

    @pl.when(cid == 1)
    def _():
      edge_loop(x2.at[1])

    plsc.subcore_barrier()

    ntrips = jnp.where(sid == NSUB - 1, (N - (NSUB - 1) * SPT) // DR,
                       SPT // DR)

    def drain(outv):
      @pl.loop(0, ntrips)
      def _(k):
        r0 = rows_base + k * DR
        pltpu.sync_copy(acc.at[pl.ds(r0, DR)], outv.at[pl.ds(r0, DR)])

    @pl.when(cid == 0)
    def _():
      drain(out2.at[0])

    @pl.when(cid == 1)
    def _():
      drain(out2.at[1])

  scratch = [
      pltpu.VMEM((NCH, K), jnp.int32),
      pltpu.VMEM((NCH, K), jnp.int32),
      pltpu.VMEM((K, FH), jnp.float32),
      pltpu.VMEM((DR, FH), jnp.float32),
      pltpu.VMEM_SHARED((NP, FH), jnp.float32),
      pltpu.SemaphoreType.DMA,
  ]
  return pl.kernel(
      body,
      out_type=jax.ShapeDtypeStruct((NCORE, N, FH), jnp.float32),
      mesh=_mesh(),
      scratch_types=scratch,
  )


BR = 2000
_NBLK = N // BR


def _tc_call(body, in_specs, out_specs, out_shape):
  return pl.pallas_call(
      body,
      grid=(_NBLK,),
      in_specs=in_specs,
      out_specs=out_specs,
      out_shape=out_shape,
  )


def _row_spec(w):
  return pl.BlockSpec((BR, w), lambda i: (i, 0))


def _split_spec():
  return pl.BlockSpec((NCORE, BR, FH), lambda i: (0, i, 0))


def _full_spec(shape):
  return pl.BlockSpec(shape, lambda i: tuple(0 for _ in shape))


def _cat(ref):
  return jnp.concatenate([ref[0], ref[1]], axis=-1)


def _prep_body(emb_ref, deg_ref, xs_ref, dinv_ref):
  dinv = lax.rsqrt(deg_ref[...] + 1.0)
  x = emb_ref[...] * dinv
  xs_ref[0] = x[:, :FH]
  xs_ref[1] = x[:, FH:]
  dinv_ref[...] = dinv


def _prep(emb, deg_c):
  return _tc_call(
      _prep_body,
      [_row_spec(F), _row_spec(1)],
      [_split_spec(), _row_spec(1)],
      (jax.ShapeDtypeStruct((NCORE, N, FH), jnp.float32),
       jax.ShapeDtypeStruct((N, 1), jnp.float32)),
  )(emb, deg_c)


def _mid_body(agg_ref, xs_ref, dinv_ref, W1_ref, b1_ref, W2_ref, ts_ref):
  dinv = dinv_ref[...]
  u = dinv * (_cat(agg_ref) + _cat(xs_ref))
  h1 = jnp.dot(u, W1_ref[...], preferred_element_type=jnp.float32) + b1_ref[...]
  t = jnp.dot(h1, W2_ref[...], preferred_element_type=jnp.float32)
  ts = t * dinv
  ts_ref[0] = ts[:, :FH]
  ts_ref[1] = ts[:, FH:]


def _mid(agg1, xs2, dinv_c, W1, b1r, W2):
  return _tc_call(
      _mid_body,
      [_split_spec(), _split_spec(), _row_spec(1),
       _full_spec((F, 2 * F)), _full_spec((1, 2 * F)), _full_spec((2 * F, F))],
      _split_spec(),
      jax.ShapeDtypeStruct((NCORE, N, FH), jnp.float32),
  )(agg1, xs2, dinv_c, W1, b1r, W2)


def _heter_body(agg_ref, ts_ref, dinv_ref, b2_ref, B_ref, ho_ref, xh_ref,
                binv_ref):
  ho = dinv_ref[...] * (_cat(agg_ref) + _cat(ts_ref)) + b2_ref[...]
  ho_ref[...] = ho
  xh_ref[0] = ho[:, :FH]
  xh_ref[1] = ho[:, FH:]
  b = B_ref[...]
  binv_ref[...] = jnp.where(b > 0.0, 1.0 / jnp.where(b > 0.0, b, 1.0), 0.0)


def _heter(agg2, ts2, dinv_c, b2r, Bc):
  return _tc_call(
      _heter_body,
      [_split_spec(), _split_spec(), _row_spec(1), _full_spec((1, F)),
       _row_spec(1)],
      [_row_spec(F), _split_spec(), _row_spec(1)],
      (jax.ShapeDtypeStruct((N, F), jnp.float32),
       jax.ShapeDtypeStruct((NCORE, N, FH), jnp.float32),
       jax.ShapeDtypeStruct((N, 1), jnp.float32)),
  )(agg2, ts2, dinv_c, b2r, Bc)


def _bscale_body(ef_ref, binv_ref, out_ref):
  binv = binv_ref[...]
  out_ref[0] = ef_ref[0] * binv
  out_ref[1] = ef_ref[1] * binv


def _bscale(ef2raw, binv_c):
  return _tc_call(
      _bscale_body,
      [_split_spec(), _row_spec(1)],
      _split_spec(),
      jax.ShapeDtypeStruct((NCORE, N, FH), jnp.float32),
  )(ef2raw, binv_c)


def _hyper_body(hp_ref, D_ref, W4_ref, b4_ref, out_ref):
  d = D_ref[...]
  dinv = jnp.where(d > 0.0, 1.0 / jnp.where(d > 0.0, d, 1.0), 0.0)
  xa = hp_ref[0] * dinv
  xb = hp_ref[1] * dinv
  W4 = W4_ref[...]
  out_ref[...] = (
      jnp.dot(xa, W4[:FH, :], preferred_element_type=jnp.float32)
      + jnp.dot(xb, W4[FH:, :], preferred_element_type=jnp.float32)
      + b4_ref[...])


def _hyper_tc(hp2, Dc, W4, b4r):
  return _tc_call(
      _hyper_body,
      [_split_spec(), _row_spec(1), _full_spec((F, F)), _full_spec((1, F))],
      _row_spec(F),
      jax.ShapeDtypeStruct((N, F), jnp.float32),
  )(hp2, Dc, W4, b4r)


_counts = _make_counts()
_agg_plain = _make_agg()


def kernel(heter_edge_index, heter_edge_type, hyper_edge_index, emb,
           W1, b1, W2, b2, W4, b4):
  del heter_edge_type
  src3 = heter_edge_index[0].reshape(NSUB, NCH, K)
  dst3 = heter_edge_index[1].reshape(NSUB, NCH, K)
  node3 = hyper_edge_index[0].reshape(NSUB, NCH, K)
  he3 = hyper_edge_index[1].reshape(NSUB, NCH, K)

  deg_p, D_p, B_p = _counts(dst3, node3, he3)
  deg_c = deg_p[:N, :1]
  Dc = D_p[:N, :1]
  Bc = B_p[:N, :1]

  xs2, dinv_c = _prep(emb, deg_c)
  agg1 = _agg_plain(xs2, src3, dst3)
  ts2 = _mid(agg1, xs2, dinv_c, W1, b1.reshape(1, -1), W2)
  agg2 = _agg_plain(ts2, src3, dst3)
  heter_out, xh2, binv_c = _heter(agg2, ts2, dinv_c, b2.reshape(1, -1), Bc)

  ef2 = _bscale(_agg_plain(xh2, node3, he3), binv_c)
  hp2 = _agg_plain(ef2, he3, node3)
  hyper_out = _hyper_tc(hp2, Dc, W4, b4.reshape(1, -1))
  return heter_out, hyper_out
